# Initial kernel scaffold; baseline (speedup 1.0000x reference)
import jax
import jax.numpy as jnp
from jax.experimental import pallas as pl


def kernel(timestamps, time_embedding):
    raise NotImplementedError("write your pallas kernel here")


# speedup vs baseline: 17.8184x; 17.8184x over previous
import functools
import math

import jax
import jax.numpy as jnp
import numpy as np
from jax.experimental import pallas as pl

_WINDOW_SIZE = 10


def _wsum(num_bins: int) -> np.ndarray:
    j = np.arange(num_bins, dtype=np.float64)[None, :]
    k = np.arange(num_bins, dtype=np.float64)[:, None]
    w = np.cos(math.pi * (j - k) / _WINDOW_SIZE) ** 2
    return np.sum(w, axis=1).astype(np.float32)


def _body(ts_ref, emb_ref, out_ref, *, wsum0, wsum1, s, d):
    ts = ts_ref[...]
    rt = (ts - ts[:, 0:1]).astype(jnp.float32)
    mx = jnp.max(rt)
    st = jnp.clip(rt / mx, 0.0, 63.0).astype(jnp.int32)
    m = (st == 1).astype(jnp.float32)
    l = s * d
    li = jax.lax.broadcasted_iota(jnp.int32, (s, l), 1)
    si = jax.lax.broadcasted_iota(jnp.int32, (s, l), 0)
    r_mat = (li // d == si).astype(jnp.float32)
    li2 = jax.lax.broadcasted_iota(jnp.int32, (d, l), 1)
    di = jax.lax.broadcasted_iota(jnp.int32, (d, l), 0)
    c_mat = (li2 % d == di).astype(jnp.float32)
    r0 = emb_ref[0:1, :] * wsum0
    r1 = emb_ref[1:2, :] * wsum1
    sel = jnp.dot(m, r_mat, preferred_element_type=jnp.float32)
    r0l = jnp.dot(r0, c_mat, preferred_element_type=jnp.float32)
    r1l = jnp.dot(r1, c_mat, preferred_element_type=jnp.float32)
    out_ref[...] = r0l * (1.0 - sel) + r1l * sel


def kernel(timestamps, time_embedding):
    b, s = timestamps.shape
    num_bins, d = time_embedding.shape
    ws = _wsum(num_bins)
    body = functools.partial(
        _body, wsum0=float(ws[0]), wsum1=float(ws[1]), s=s, d=d
    )
    out2 = pl.pallas_call(
        body,
        out_shape=jax.ShapeDtypeStruct((b, s * d), jnp.float32),
    )(timestamps, time_embedding)
    return out2.reshape(b, s, d)
